# Initial kernel scaffold; baseline (speedup 1.0000x reference)
import jax
import jax.numpy as jnp
from jax.experimental import pallas as pl


def kernel(x, edge_index, edge_weight, W1, b1, g1, be1, W2, b2, g2, be2, W3, b3, g3, be3, W4, b4, g4, be4, W5, b5, g5, be5):
    raise NotImplementedError("write your pallas kernel here")


# speedup vs baseline: 6.6878x; 6.6878x over previous
import functools
import jax
import jax.numpy as jnp
from jax import lax
from jax.experimental import pallas as pl
from jax.experimental.pallas import tpu as pltpu
from jax.experimental.pallas import tpu_sc as plsc

N = 10000
E = 320000
NC = 2
NS = 16
NW = NC * NS
EPT = E // NW
KE = 80
NCHUNK = EPT // KE
NPAD = 10240
ZROWS = NPAD // NS
WTILES = 10
WROWS = N // WTILES

_EPS = 1e-5


def _make_agg(C):
    mesh = plsc.VectorSubcoreMesh(core_axis_name="c", subcore_axis_name="s")

    @functools.partial(
        pl.kernel,
        mesh=mesh,
        out_type=jax.ShapeDtypeStruct((NC, N, C), jnp.float32),
        scratch_types=[
            pltpu.VMEM((KE,), jnp.int32),
            pltpu.VMEM((KE,), jnp.int32),
            pltpu.VMEM((KE,), jnp.float32),
            pltpu.VMEM((KE, C), jnp.float32),
            pltpu.VMEM_SHARED((NPAD, C), jnp.float32),
            pltpu.SemaphoreType.DMA,
        ],
        compiler_params=pltpu.CompilerParams(use_tc_tiling_on_sc=False),
    )
    def agg(y_hbm, src_hbm, dst_hbm, ew_hbm, zeros_hbm, out_hbm,
            src_v, dst_v, ew_v, rows_v, acc_sh, gsem):
        cid = lax.axis_index("c")
        sid = lax.axis_index("s")

        z0 = sid * ZROWS
        pltpu.sync_copy(zeros_hbm.at[pl.ds(z0, ZROWS)],
                        acc_sh.at[pl.ds(z0, ZROWS)])
        plsc.subcore_barrier()

        ebase = (sid * NC + cid) * EPT

        def chunk(j, carry):
            e0 = ebase + j * KE
            pltpu.sync_copy(src_hbm.at[pl.ds(e0, KE)], src_v)
            pltpu.sync_copy(dst_hbm.at[pl.ds(e0, KE)], dst_v)
            pltpu.sync_copy(ew_hbm.at[pl.ds(e0, KE)], ew_v)
            pltpu.async_copy(y_hbm.at[src_v], rows_v, gsem).wait()

            def group(g, c2):
                ew16 = ew_v[pl.ds(g * 16, 16)]
                for j in range(16):
                    s = ew16[j]
                    k = g * 16 + j
                    for c in range(C // 16):
                        rows_v[k, pl.ds(c * 16, 16)] = (
                            rows_v[k, pl.ds(c * 16, 16)] * s)
                return c2
            lax.fori_loop(0, KE // 16, group, 0)

            pltpu.sync_copy(rows_v, acc_sh.at[dst_v], add=True)
            return carry

        lax.fori_loop(0, NCHUNK, chunk, 0)
        plsc.subcore_barrier()

        @pl.when(sid < WTILES)
        def _():
            w0 = sid * WROWS
            pltpu.sync_copy(acc_sh.at[pl.ds(w0, WROWS)],
                            out_hbm.at[cid, pl.ds(w0, WROWS)])

    return agg


_agg16 = _make_agg(16)
_agg64 = _make_agg(64)
_agg128 = _make_agg(128)


def _agg_call(y, src, dst, ew, zeros):
    C = y.shape[1]
    fn = {16: _agg16, 64: _agg64, 128: _agg128}[C]
    return fn(y, src, dst, ew, zeros)


def _dot(a, b):
    return lax.dot_general(a, b, (((1,), (0,)), ((), ())),
                           precision=lax.Precision.HIGHEST,
                           preferred_element_type=jnp.float32)


def _bn(h, g, be):
    m = jnp.mean(h, axis=0, keepdims=True)
    d = h - m
    v = jnp.mean(d * d, axis=0, keepdims=True)
    return d * lax.rsqrt(v + _EPS) * g + be


def _tc_call(body, out_shapes, *args):
    return pl.pallas_call(
        body,
        out_shape=[jax.ShapeDtypeStruct(s, jnp.float32) for s in out_shapes],
        compiler_params=pltpu.CompilerParams(vmem_limit_bytes=100 * 2**20),
    )(*args)


_BR = 2000


def _row_call(body, out_cols, *args):
    grid = (N // _BR,)
    in_specs = []
    for a in args:
        if a.shape[0] == N:
            in_specs.append(
                pl.BlockSpec((_BR, a.shape[1]), lambda i: (i, 0)))
        else:
            in_specs.append(
                pl.BlockSpec(a.shape, lambda i, nd=a.ndim: (0,) * nd))
    out_specs = [pl.BlockSpec((_BR, c), lambda i: (i, 0)) for c in out_cols]
    return pl.pallas_call(
        body, grid=grid, in_specs=in_specs, out_specs=out_specs,
        out_shape=[jax.ShapeDtypeStruct((N, c), jnp.float32)
                   for c in out_cols],
    )(*args)


def _tc1(dega_ref, degb_ref, x_ref, w1_ref, dinv_ref, u1_ref):
    deg = dega_ref[:, :] + degb_ref[:, :] + 1.0
    dinv = lax.rsqrt(deg)
    dinv_ref[:, :] = dinv
    u1_ref[:, :] = dinv[:, 0:1] * _dot(x_ref[:, :], w1_ref[:, :])


def _tc2(sa_ref, sb_ref, u_ref, dinv_ref, g_ref, be_ref, u2_ref):
    dc = dinv_ref[:, 0:1]
    t = dc * (sa_ref[:, :] + sb_ref[:, :] + u_ref[:, :])
    h = jnp.maximum(_bn(t, g_ref[:, :], be_ref[:, :]), 0.0)
    u2_ref[:, :] = dc * h


def _tc3(sa_ref, sb_ref, u_ref, dinv_ref, w_ref, g_ref, be_ref, u3_ref):
    dc = dinv_ref[:, 0:1]
    t = dc * (sa_ref[:, :] + sb_ref[:, :] + u_ref[:, :])
    h = jnp.maximum(_bn(_dot(t, w_ref[:, :]), g_ref[:, :], be_ref[:, :]), 0.0)
    u3_ref[:, :] = dc * h


def _add3(sa_ref, sb_ref, u_ref, dinv_ref, t_ref):
    dc = dinv_ref[:, 0:1]
    t_ref[:, :] = dc * (sa_ref[:, :] + sb_ref[:, :] + u_ref[:, :])


def _mmbnrelu(t_ref, w_ref, g_ref, be_ref, h_ref):
    h_ref[:, :] = jnp.maximum(
        _bn(_dot(t_ref[:, :], w_ref[:, :]), g_ref[:, :], be_ref[:, :]), 0.0)


def _tc4b(h3_ref, dinv_ref, w4_ref, u4a_ref, u4b_ref):
    dc = dinv_ref[:, 0:1]
    u4 = dc * _dot(h3_ref[:, :], w4_ref[:, :])
    u4a_ref[:, :] = u4[:, 0:128]
    u4b_ref[:, :] = u4[:, 128:256]


def _tc5(ta_ref, tb_ref, ga_ref, bea_ref, gb_ref, beb_ref, w5a_ref, w5b_ref,
         dinv_ref, u5_ref):
    h4a = jnp.maximum(_bn(ta_ref[:, :], ga_ref[:, :], bea_ref[:, :]), 0.0)
    h4b = jnp.maximum(_bn(tb_ref[:, :], gb_ref[:, :], beb_ref[:, :]), 0.0)
    u5 = _dot(h4a, w5a_ref[:, :]) + _dot(h4b, w5b_ref[:, :])
    u5_ref[:, :] = dinv_ref[:, 0:1] * u5


def _tc6(sa_ref, sb_ref, u_ref, dinv_ref, g_ref, be_ref, out_ref):
    dc = dinv_ref[:, 0:1]
    t = dc * (sa_ref[:, :] + sb_ref[:, :] + u_ref[:, :])
    out_ref[:, :] = _bn(t, g_ref[:, :], be_ref[:, :])


@jax.jit
def kernel(x, edge_index, edge_weight,
           W1, b1, g1, be1, W2, b2, g2, be2, W3, b3, g3, be3,
           W4, b4, g4, be4, W5, b5, g5, be5):
    src = edge_index[0]
    dst = edge_index[1]
    ew = edge_weight

    zeros128 = jnp.zeros((NPAD, 128), jnp.float32)
    ones16 = jnp.ones((N, 16), jnp.float32)

    row2 = lambda a: a.reshape(1, -1)

    degp = _agg_call(ones16, src, dst, ew, zeros128[:, :16])

    dinv, u1 = _row_call(_tc1, [16, 64], degp[0], degp[1], x, W1)

    s1 = _agg_call(u1, src, dst, ew, zeros128[:, :64])
    (u2,) = _tc_call(_tc2, [(N, 64)],
                     s1[0], s1[1], u1, dinv, row2(g1), row2(be1))

    s2 = _agg_call(u2, src, dst, ew, zeros128[:, :64])
    (u3,) = _tc_call(_tc3, [(N, 128)],
                     s2[0], s2[1], u2, dinv, W2, row2(g2), row2(be2))

    s3 = _agg_call(u3, src, dst, ew, zeros128)
    (t3,) = _row_call(_add3, [128], s3[0], s3[1], u3, dinv)
    (h3,) = _tc_call(_mmbnrelu, [(N, 256)], t3, W3, row2(g3), row2(be3))
    u4a, u4b = _row_call(_tc4b, [128, 128], h3, dinv, W4)

    s4a = _agg_call(u4a, src, dst, ew, zeros128)
    s4b = _agg_call(u4b, src, dst, ew, zeros128)
    (t4a,) = _row_call(_add3, [128], s4a[0], s4a[1], u4a, dinv)
    (t4b,) = _row_call(_add3, [128], s4b[0], s4b[1], u4b, dinv)
    g4r, be4r = row2(g4), row2(be4)
    (u5,) = _tc_call(_tc5, [(N, 128)],
                     t4a, t4b, g4r[:, :128], be4r[:, :128],
                     g4r[:, 128:], be4r[:, 128:],
                     W5[:128], W5[128:], dinv)

    s5 = _agg_call(u5, src, dst, ew, zeros128)
    (out,) = _tc_call(_tc6, [(N, 128)],
                      s5[0], s5[1], u5, dinv, row2(g5), row2(be5))
    return out


# speedup vs baseline: 17.3702x; 2.5973x over previous
import functools
import jax
import jax.numpy as jnp
from jax import lax
from jax.experimental import pallas as pl
from jax.experimental.pallas import tpu as pltpu
from jax.experimental.pallas import tpu_sc as plsc

N = 10000
E = 320000
NC = 2
NS = 16
NW = NC * NS
EPT = E // NW
KE = 80
NCHUNK = EPT // KE
NPAD = 10240
ZROWS = NPAD // NS
WTILES = 10
WROWS = N // WTILES

_EPS = 1e-5


def _make_agg(C):
    mesh = plsc.VectorSubcoreMesh(core_axis_name="c", subcore_axis_name="s")

    @functools.partial(
        pl.kernel,
        mesh=mesh,
        out_type=jax.ShapeDtypeStruct((NC, N, C), jnp.float32),
        scratch_types=[
            pltpu.VMEM((EPT,), jnp.int32),
            pltpu.VMEM((NCHUNK, KE), jnp.int32),
            pltpu.VMEM((KE,), jnp.float32),
            pltpu.VMEM((KE,), jnp.float32),
            pltpu.VMEM((KE, C), jnp.float32),
            pltpu.VMEM((KE, C), jnp.float32),
            pltpu.VMEM_SHARED((NPAD, C), jnp.float32),
            pltpu.SemaphoreType.DMA,
            pltpu.SemaphoreType.DMA,
            pltpu.SemaphoreType.DMA,
            pltpu.SemaphoreType.DMA,
        ],
        compiler_params=pltpu.CompilerParams(use_tc_tiling_on_sc=False),
    )
    def agg(y_hbm, src_hbm, dst3_hbm, ew_hbm, zeros_hbm, out_hbm,
            src_v, dst_v, ewb0, ewb1, rows0, rows1, acc_sh,
            gsem0, gsem1, esem0, esem1):
        cid = lax.axis_index("c")
        sid = lax.axis_index("s")
        w = sid * NC + cid
        rows = (rows0, rows1)
        ewb = (ewb0, ewb1)
        gsems = (gsem0, gsem1)
        esems = (esem0, esem1)

        z0 = sid * ZROWS
        pltpu.sync_copy(zeros_hbm.at[pl.ds(z0, ZROWS)],
                        acc_sh.at[pl.ds(z0, ZROWS)])
        pltpu.sync_copy(src_hbm.at[pl.ds(w * EPT, EPT)], src_v)
        pltpu.sync_copy(dst3_hbm.at[w], dst_v)
        plsc.subcore_barrier()

        def gcopy(j, b):
            return pltpu.make_async_copy(
                y_hbm.at[src_v.at[pl.ds(j * KE, KE)]], rows[b], gsems[b])

        def ecopy(j, b):
            return pltpu.make_async_copy(
                ew_hbm.at[pl.ds(w * EPT + j * KE, KE)], ewb[b], esems[b])

        def start(j, b):
            ecopy(j, b).start()
            gcopy(j, b).start()

        def scale(b):
            def group(g, c2):
                ew16 = ewb[b][pl.ds(g * 16, 16)]
                for t in range(16):
                    s = ew16[t]
                    k = g * 16 + t
                    for c in range(C // 16):
                        rows[b][k, pl.ds(c * 16, 16)] = (
                            rows[b][k, pl.ds(c * 16, 16)] * s)
                return c2
            lax.fori_loop(0, KE // 16, group, 0)

        def process(j, b):
            gcopy(j, b).wait()
            ecopy(j, b).wait()
            scale(b)
            pltpu.sync_copy(rows[b], acc_sh.at[dst_v.at[j]], add=True)

        start(0, 0)
        start(1, 1)

        def main_body(i, carry):
            for b in range(2):
                j = i * 2 + b
                process(j, b)
                start(j + 2, b)
            return carry
        lax.fori_loop(0, (NCHUNK - 3) // 2, main_body, 0)

        process(NCHUNK - 3, 0)
        start(NCHUNK - 1, 0)
        process(NCHUNK - 2, 1)
        process(NCHUNK - 1, 0)
        plsc.subcore_barrier()

        @pl.when(sid < WTILES)
        def _():
            w0 = sid * WROWS
            pltpu.sync_copy(acc_sh.at[pl.ds(w0, WROWS)],
                            out_hbm.at[cid, pl.ds(w0, WROWS)])

    return agg


_agg16 = _make_agg(16)
_agg64 = _make_agg(64)
_agg128 = _make_agg(128)


def _agg_call(y, src, dst, ew, zeros):
    C = y.shape[1]
    fn = {16: _agg16, 64: _agg64, 128: _agg128}[C]
    return fn(y, src, dst, ew, zeros)


def _dot(a, b):
    return lax.dot_general(a, b, (((1,), (0,)), ((), ())),
                           precision=lax.Precision.HIGHEST,
                           preferred_element_type=jnp.float32)


def _bn(h, g, be):
    m = jnp.mean(h, axis=0, keepdims=True)
    d = h - m
    v = jnp.mean(d * d, axis=0, keepdims=True)
    return d * lax.rsqrt(v + _EPS) * g + be


def _tc_call(body, out_shapes, *args):
    return pl.pallas_call(
        body,
        out_shape=[jax.ShapeDtypeStruct(s, jnp.float32) for s in out_shapes],
        compiler_params=pltpu.CompilerParams(vmem_limit_bytes=100 * 2**20),
    )(*args)


_BR = 2000


def _row_call(body, out_cols, *args):
    grid = (N // _BR,)
    in_specs = []
    for a in args:
        if a.shape[0] == N:
            in_specs.append(
                pl.BlockSpec((_BR, a.shape[1]), lambda i: (i, 0)))
        else:
            in_specs.append(
                pl.BlockSpec(a.shape, lambda i, nd=a.ndim: (0,) * nd))
    out_specs = [pl.BlockSpec((_BR, c), lambda i: (i, 0)) for c in out_cols]
    return pl.pallas_call(
        body, grid=grid, in_specs=in_specs, out_specs=out_specs,
        out_shape=[jax.ShapeDtypeStruct((N, c), jnp.float32)
                   for c in out_cols],
    )(*args)


def _tc1(dega_ref, degb_ref, x_ref, w1_ref, dinv_ref, u1_ref):
    deg = dega_ref[:, :] + degb_ref[:, :] + 1.0
    dinv = lax.rsqrt(deg)
    dinv_ref[:, :] = dinv
    u1_ref[:, :] = dinv[:, 0:1] * _dot(x_ref[:, :], w1_ref[:, :])


def _tc2(sa_ref, sb_ref, u_ref, dinv_ref, g_ref, be_ref, u2_ref):
    dc = dinv_ref[:, 0:1]
    t = dc * (sa_ref[:, :] + sb_ref[:, :] + u_ref[:, :])
    h = jnp.maximum(_bn(t, g_ref[:, :], be_ref[:, :]), 0.0)
    u2_ref[:, :] = dc * h


def _tc3(sa_ref, sb_ref, u_ref, dinv_ref, w_ref, g_ref, be_ref, u3_ref):
    dc = dinv_ref[:, 0:1]
    t = dc * (sa_ref[:, :] + sb_ref[:, :] + u_ref[:, :])
    h = jnp.maximum(_bn(_dot(t, w_ref[:, :]), g_ref[:, :], be_ref[:, :]), 0.0)
    u3_ref[:, :] = dc * h


def _add3(sa_ref, sb_ref, u_ref, dinv_ref, t_ref):
    dc = dinv_ref[:, 0:1]
    t_ref[:, :] = dc * (sa_ref[:, :] + sb_ref[:, :] + u_ref[:, :])


def _mmbnrelu(t_ref, w_ref, g_ref, be_ref, h_ref):
    h_ref[:, :] = jnp.maximum(
        _bn(_dot(t_ref[:, :], w_ref[:, :]), g_ref[:, :], be_ref[:, :]), 0.0)


def _tc4b(h3_ref, dinv_ref, w4_ref, u4a_ref, u4b_ref):
    dc = dinv_ref[:, 0:1]
    u4 = dc * _dot(h3_ref[:, :], w4_ref[:, :])
    u4a_ref[:, :] = u4[:, 0:128]
    u4b_ref[:, :] = u4[:, 128:256]


def _tc5(ta_ref, tb_ref, ga_ref, bea_ref, gb_ref, beb_ref, w5a_ref, w5b_ref,
         dinv_ref, u5_ref):
    h4a = jnp.maximum(_bn(ta_ref[:, :], ga_ref[:, :], bea_ref[:, :]), 0.0)
    h4b = jnp.maximum(_bn(tb_ref[:, :], gb_ref[:, :], beb_ref[:, :]), 0.0)
    u5 = _dot(h4a, w5a_ref[:, :]) + _dot(h4b, w5b_ref[:, :])
    u5_ref[:, :] = dinv_ref[:, 0:1] * u5


def _tc6(sa_ref, sb_ref, u_ref, dinv_ref, g_ref, be_ref, out_ref):
    dc = dinv_ref[:, 0:1]
    t = dc * (sa_ref[:, :] + sb_ref[:, :] + u_ref[:, :])
    out_ref[:, :] = _bn(t, g_ref[:, :], be_ref[:, :])


@jax.jit
def kernel(x, edge_index, edge_weight,
           W1, b1, g1, be1, W2, b2, g2, be2, W3, b3, g3, be3,
           W4, b4, g4, be4, W5, b5, g5, be5):
    src = edge_index[0]
    dst = edge_index[1].reshape(NW, NCHUNK, KE)
    ew = edge_weight

    zeros128 = jnp.zeros((NPAD, 128), jnp.float32)
    ones16 = jnp.ones((N, 16), jnp.float32)

    row2 = lambda a: a.reshape(1, -1)

    degp = _agg_call(ones16, src, dst, ew, zeros128[:, :16])

    dinv, u1 = _row_call(_tc1, [16, 64], degp[0], degp[1], x, W1)

    s1 = _agg_call(u1, src, dst, ew, zeros128[:, :64])
    (u2,) = _tc_call(_tc2, [(N, 64)],
                     s1[0], s1[1], u1, dinv, row2(g1), row2(be1))

    s2 = _agg_call(u2, src, dst, ew, zeros128[:, :64])
    (u3,) = _tc_call(_tc3, [(N, 128)],
                     s2[0], s2[1], u2, dinv, W2, row2(g2), row2(be2))

    s3 = _agg_call(u3, src, dst, ew, zeros128)
    (t3,) = _row_call(_add3, [128], s3[0], s3[1], u3, dinv)
    (h3,) = _tc_call(_mmbnrelu, [(N, 256)], t3, W3, row2(g3), row2(be3))
    u4a, u4b = _row_call(_tc4b, [128, 128], h3, dinv, W4)

    s4a = _agg_call(u4a, src, dst, ew, zeros128)
    s4b = _agg_call(u4b, src, dst, ew, zeros128)
    (t4a,) = _row_call(_add3, [128], s4a[0], s4a[1], u4a, dinv)
    (t4b,) = _row_call(_add3, [128], s4b[0], s4b[1], u4b, dinv)
    g4r, be4r = row2(g4), row2(be4)
    (u5,) = _tc_call(_tc5, [(N, 128)],
                     t4a, t4b, g4r[:, :128], be4r[:, :128],
                     g4r[:, 128:], be4r[:, 128:],
                     W5[:128], W5[128:], dinv)

    s5 = _agg_call(u5, src, dst, ew, zeros128)
    (out,) = _tc_call(_tc6, [(N, 128)],
                      s5[0], s5[1], u5, dinv, row2(g5), row2(be5))
    return out


# speedup vs baseline: 19.2159x; 1.1063x over previous
import functools
import jax
import jax.numpy as jnp
from jax import lax
from jax.experimental import pallas as pl
from jax.experimental.pallas import tpu as pltpu
from jax.experimental.pallas import tpu_sc as plsc

N = 10000
E = 320000
NC = 2
NS = 16
NW = NC * NS
EPT = E // NW
KE = 80
NCHUNK = EPT // KE
WTILES = 10
WROWS = N // WTILES

_EPS = 1e-5


def _make_agg(C):
    mesh = plsc.VectorSubcoreMesh(core_axis_name="c", subcore_axis_name="s")

    @functools.partial(
        pl.kernel,
        mesh=mesh,
        out_type=jax.ShapeDtypeStruct((NC, N, C), jnp.float32),
        scratch_types=[
            pltpu.VMEM((EPT,), jnp.int32),
            pltpu.VMEM((NCHUNK, KE), jnp.int32),
            pltpu.VMEM((KE,), jnp.float32),
            pltpu.VMEM((KE,), jnp.float32),
            pltpu.VMEM((KE,), jnp.float32),
            pltpu.VMEM((KE, C), jnp.float32),
            pltpu.VMEM((KE, C), jnp.float32),
            pltpu.VMEM((KE, C), jnp.float32),
            pltpu.VMEM_SHARED((N, C), jnp.float32),
            pltpu.SemaphoreType.DMA,
            pltpu.SemaphoreType.DMA,
            pltpu.SemaphoreType.DMA,
            pltpu.SemaphoreType.DMA,
            pltpu.SemaphoreType.DMA,
            pltpu.SemaphoreType.DMA,
            pltpu.SemaphoreType.DMA,
            pltpu.SemaphoreType.DMA,
            pltpu.SemaphoreType.DMA,
        ],
        compiler_params=pltpu.CompilerParams(use_tc_tiling_on_sc=False),
    )
    def agg(y_hbm, src_hbm, dst3_hbm, ew_hbm, zeros_hbm, out_hbm,
            src_v, dst_v, ewb0, ewb1, ewb2, r0, r1, r2, acc_sh,
            g0, g1, g2, e0, e1, e2, s0, s1, s2):
        cid = lax.axis_index("c")
        sid = lax.axis_index("s")
        w = sid * NC + cid
        rows = (r0, r1, r2)
        ewb = (ewb0, ewb1, ewb2)
        gsems = (g0, g1, g2)
        esems = (e0, e1, e2)
        ssems = (s0, s1, s2)

        @pl.when(sid < WTILES)
        def _():
            z0 = sid * WROWS
            pltpu.sync_copy(zeros_hbm.at[pl.ds(z0, WROWS)],
                            acc_sh.at[pl.ds(z0, WROWS)])
        pltpu.sync_copy(src_hbm.at[pl.ds(w * EPT, EPT)], src_v)
        pltpu.sync_copy(dst3_hbm.at[w], dst_v)
        plsc.subcore_barrier()

        def gcopy(j, b):
            return pltpu.make_async_copy(
                y_hbm.at[src_v.at[pl.ds(j * KE, KE)]], rows[b], gsems[b])

        def ecopy(j, b):
            return pltpu.make_async_copy(
                ew_hbm.at[pl.ds(w * EPT + j * KE, KE)], ewb[b], esems[b])

        def scopy(j, b):
            return pltpu.make_async_copy(
                rows[b], acc_sh.at[dst_v.at[j]], ssems[b])

        def start(j, b3):
            ecopy(j, b3).start()
            gcopy(j, b3).start()

        def scale(b3):
            def group(g, c2):
                ew16 = ewb[b3][pl.ds(g * 16, 16)]
                for t in range(16):
                    s = ew16[t]
                    k = g * 16 + t
                    for c in range(C // 16):
                        rows[b3][k, pl.ds(c * 16, 16)] = (
                            rows[b3][k, pl.ds(c * 16, 16)] * s)
                return c2
            lax.fori_loop(0, KE // 16, group, 0)

        def step(j, b3, first=False, last=False):
            gcopy(j, b3).wait()
            ecopy(j, b3).wait()
            scale(b3)
            pltpu.async_copy(rows[b3], acc_sh.at[dst_v.at[j]], ssems[b3],
                             add=True)
            if not first:
                scopy(j - 1, (b3 - 1) % 3).wait()
            if not last:
                start(j + 2, (b3 + 2) % 3)

        start(0, 0)
        start(1, 1)
        step(0, 0, first=True)
        step(1, 1)

        def main_body(i, carry):
            for r in range(3):
                j = 3 * i + 2 + r
                step(j, (2 + r) % 3)
            return carry
        lax.fori_loop(0, (NCHUNK - 5) // 3, main_body, 0)

        step(NCHUNK - 3, (NCHUNK - 3) % 3)
        step(NCHUNK - 2, (NCHUNK - 2) % 3, last=True)
        step(NCHUNK - 1, (NCHUNK - 1) % 3, last=True)
        scopy(NCHUNK - 1, (NCHUNK - 1) % 3).wait()
        plsc.subcore_barrier()

        @pl.when(sid < WTILES)
        def _():
            w0 = sid * WROWS
            pltpu.sync_copy(acc_sh.at[pl.ds(w0, WROWS)],
                            out_hbm.at[cid, pl.ds(w0, WROWS)])

    return agg


_agg16 = _make_agg(16)
_agg64 = _make_agg(64)
_agg128 = _make_agg(128)


def _agg_call(y, src, dst, ew, zeros):
    C = y.shape[1]
    fn = {16: _agg16, 64: _agg64, 128: _agg128}[C]
    return fn(y, src, dst, ew, zeros)


def _dot(a, b):
    return lax.dot_general(a, b, (((1,), (0,)), ((), ())),
                           precision=lax.Precision.HIGHEST,
                           preferred_element_type=jnp.float32)


def _bn(h, g, be):
    m = jnp.mean(h, axis=0, keepdims=True)
    d = h - m
    v = jnp.mean(d * d, axis=0, keepdims=True)
    return d * lax.rsqrt(v + _EPS) * g + be


def _tc_call(body, out_shapes, *args):
    return pl.pallas_call(
        body,
        out_shape=[jax.ShapeDtypeStruct(s, jnp.float32) for s in out_shapes],
        compiler_params=pltpu.CompilerParams(vmem_limit_bytes=100 * 2**20),
    )(*args)


_BR = 2000


def _row_call(body, out_cols, *args):
    grid = (N // _BR,)
    in_specs = []
    for a in args:
        if a.shape[0] == N:
            in_specs.append(
                pl.BlockSpec((_BR, a.shape[1]), lambda i: (i, 0)))
        else:
            in_specs.append(
                pl.BlockSpec(a.shape, lambda i, nd=a.ndim: (0,) * nd))
    out_specs = [pl.BlockSpec((_BR, c), lambda i: (i, 0)) for c in out_cols]
    return pl.pallas_call(
        body, grid=grid, in_specs=in_specs, out_specs=out_specs,
        out_shape=[jax.ShapeDtypeStruct((N, c), jnp.float32)
                   for c in out_cols],
    )(*args)


def _tc1(dega_ref, degb_ref, x_ref, w1_ref, dinv_ref, u1_ref):
    deg = dega_ref[:, :] + degb_ref[:, :] + 1.0
    dinv = lax.rsqrt(deg)
    dinv_ref[:, :] = dinv
    u1_ref[:, :] = dinv[:, 0:1] * _dot(x_ref[:, :], w1_ref[:, :])


def _tc2(sa_ref, sb_ref, u_ref, dinv_ref, g_ref, be_ref, u2_ref):
    dc = dinv_ref[:, 0:1]
    t = dc * (sa_ref[:, :] + sb_ref[:, :] + u_ref[:, :])
    h = jnp.maximum(_bn(t, g_ref[:, :], be_ref[:, :]), 0.0)
    u2_ref[:, :] = dc * h


def _tc3(sa_ref, sb_ref, u_ref, dinv_ref, w_ref, g_ref, be_ref, u3_ref):
    dc = dinv_ref[:, 0:1]
    t = dc * (sa_ref[:, :] + sb_ref[:, :] + u_ref[:, :])
    h = jnp.maximum(_bn(_dot(t, w_ref[:, :]), g_ref[:, :], be_ref[:, :]), 0.0)
    u3_ref[:, :] = dc * h


def _add3(sa_ref, sb_ref, u_ref, dinv_ref, t_ref):
    dc = dinv_ref[:, 0:1]
    t_ref[:, :] = dc * (sa_ref[:, :] + sb_ref[:, :] + u_ref[:, :])


def _mmbnrelu(t_ref, w_ref, g_ref, be_ref, h_ref):
    h_ref[:, :] = jnp.maximum(
        _bn(_dot(t_ref[:, :], w_ref[:, :]), g_ref[:, :], be_ref[:, :]), 0.0)


def _tc4b(h3_ref, dinv_ref, w4_ref, u4a_ref, u4b_ref):
    dc = dinv_ref[:, 0:1]
    u4 = dc * _dot(h3_ref[:, :], w4_ref[:, :])
    u4a_ref[:, :] = u4[:, 0:128]
    u4b_ref[:, :] = u4[:, 128:256]


def _tc5(ta_ref, tb_ref, ga_ref, bea_ref, gb_ref, beb_ref, w5a_ref, w5b_ref,
         dinv_ref, u5_ref):
    h4a = jnp.maximum(_bn(ta_ref[:, :], ga_ref[:, :], bea_ref[:, :]), 0.0)
    h4b = jnp.maximum(_bn(tb_ref[:, :], gb_ref[:, :], beb_ref[:, :]), 0.0)
    u5 = _dot(h4a, w5a_ref[:, :]) + _dot(h4b, w5b_ref[:, :])
    u5_ref[:, :] = dinv_ref[:, 0:1] * u5


def _tc6(sa_ref, sb_ref, u_ref, dinv_ref, g_ref, be_ref, out_ref):
    dc = dinv_ref[:, 0:1]
    t = dc * (sa_ref[:, :] + sb_ref[:, :] + u_ref[:, :])
    out_ref[:, :] = _bn(t, g_ref[:, :], be_ref[:, :])


@jax.jit
def kernel(x, edge_index, edge_weight,
           W1, b1, g1, be1, W2, b2, g2, be2, W3, b3, g3, be3,
           W4, b4, g4, be4, W5, b5, g5, be5):
    src = edge_index[0]
    dst = edge_index[1].reshape(NW, NCHUNK, KE)
    ew = edge_weight

    zeros128 = jnp.zeros((N, 128), jnp.float32)
    ones16 = jnp.ones((N, 16), jnp.float32)

    row2 = lambda a: a.reshape(1, -1)

    degp = _agg_call(ones16, src, dst, ew, zeros128[:, :16])

    dinv, u1 = _row_call(_tc1, [16, 64], degp[0], degp[1], x, W1)

    s1 = _agg_call(u1, src, dst, ew, zeros128[:, :64])
    (u2,) = _tc_call(_tc2, [(N, 64)],
                     s1[0], s1[1], u1, dinv, row2(g1), row2(be1))

    s2 = _agg_call(u2, src, dst, ew, zeros128[:, :64])
    (u3,) = _tc_call(_tc3, [(N, 128)],
                     s2[0], s2[1], u2, dinv, W2, row2(g2), row2(be2))

    s3 = _agg_call(u3, src, dst, ew, zeros128)
    (t3,) = _row_call(_add3, [128], s3[0], s3[1], u3, dinv)
    (h3,) = _tc_call(_mmbnrelu, [(N, 256)], t3, W3, row2(g3), row2(be3))
    u4a, u4b = _row_call(_tc4b, [128, 128], h3, dinv, W4)

    s4a = _agg_call(u4a, src, dst, ew, zeros128)
    s4b = _agg_call(u4b, src, dst, ew, zeros128)
    (t4a,) = _row_call(_add3, [128], s4a[0], s4a[1], u4a, dinv)
    (t4b,) = _row_call(_add3, [128], s4b[0], s4b[1], u4b, dinv)
    g4r, be4r = row2(g4), row2(be4)
    (u5,) = _tc_call(_tc5, [(N, 128)],
                     t4a, t4b, g4r[:, :128], be4r[:, :128],
                     g4r[:, 128:], be4r[:, 128:],
                     W5[:128], W5[128:], dinv)

    s5 = _agg_call(u5, src, dst, ew, zeros128)
    (out,) = _tc_call(_tc6, [(N, 128)],
                      s5[0], s5[1], u5, dinv, row2(g5), row2(be5))
    return out
